# baseline (speedup 1.0000x reference)
import jax
import jax.numpy as jnp
from jax.experimental import pallas as pl
from jax.experimental.pallas import tpu as pltpu

N = 10000
K = 32
D_IN = 128
D_OUT = 128

NH = N // 2
BH = 200


def _body(self_ref, n0_ref, n1_ref, wts_ref, wtn_ref, b_ref, out_ref):
    wts = wts_ref[...]
    wtn = wtn_ref[...]
    b = b_ref[...]
    for half, n_ref in ((0, n0_ref), (1, n1_ref)):
        m = jnp.sum(n_ref[0], axis=1) * (1.0 / K)
        t = jnp.dot(self_ref[half], wts, preferred_element_type=jnp.float32)
        t = t + jnp.dot(m, wtn, preferred_element_type=jnp.float32)
        t = t + b
        c = jnp.maximum(t, 0.0)
        norm2 = jnp.sum(c * c, axis=1, keepdims=True)
        out_ref[half] = c * jax.lax.rsqrt(jnp.maximum(norm2, 1e-24))


def kernel(self_embs, neigh_embs, W_self, b_self, W_neigh, b_neigh):
    wts = W_self.T
    wtn = W_neigh.T
    b = (b_self + b_neigh).reshape(1, D_OUT)
    sr = self_embs.reshape(2, NH, D_IN)
    nr = neigh_embs.reshape(2, NH, K, D_IN)
    out = pl.pallas_call(
        _body,
        grid=(NH // BH,),
        in_specs=[
            pl.BlockSpec((2, BH, D_IN), lambda i: (0, i, 0)),
            pl.BlockSpec((1, BH, K, D_IN), lambda i: (0, i, 0, 0)),
            pl.BlockSpec((1, BH, K, D_IN), lambda i: (1, i, 0, 0)),
            pl.BlockSpec((D_IN, D_OUT), lambda i: (0, 0)),
            pl.BlockSpec((D_IN, D_OUT), lambda i: (0, 0)),
            pl.BlockSpec((1, D_OUT), lambda i: (0, 0)),
        ],
        out_specs=pl.BlockSpec((2, BH, D_OUT), lambda i: (0, i, 0)),
        out_shape=jax.ShapeDtypeStruct((2, NH, D_OUT), jnp.float32),
        compiler_params=pltpu.CompilerParams(
            dimension_semantics=("arbitrary",),
        ),
    )(sr, nr, nr, wts, wtn, b)
    return out.reshape(N, D_OUT)


# speedup vs baseline: 1.0048x; 1.0048x over previous
import jax
import jax.numpy as jnp
from jax.experimental import pallas as pl
from jax.experimental.pallas import tpu as pltpu

N = 10000
K = 32
D_IN = 128
D_OUT = 128

BN = 400


def _body(self_ref, neigh_ref, wts_ref, wtn_ref, b_ref, out_ref):
    neigh_mean = jnp.sum(neigh_ref[...], axis=1) * (1.0 / K)
    t = jnp.dot(self_ref[...], wts_ref[...], preferred_element_type=jnp.float32)
    t = t + jnp.dot(neigh_mean, wtn_ref[...], preferred_element_type=jnp.float32)
    t = t + b_ref[...]
    c = jnp.maximum(t, 0.0)
    norm2 = jnp.sum(c * c, axis=1, keepdims=True)
    out_ref[...] = c * jax.lax.rsqrt(jnp.maximum(norm2, 1e-24))


def kernel(self_embs, neigh_embs, W_self, b_self, W_neigh, b_neigh):
    wts = W_self.T
    wtn = W_neigh.T
    b = (b_self + b_neigh).reshape(1, D_OUT)
    return pl.pallas_call(
        _body,
        grid=(N // BN,),
        in_specs=[
            pl.BlockSpec((BN, D_IN), lambda i: (i, 0)),
            pl.BlockSpec((BN, K, D_IN), lambda i: (i, 0, 0)),
            pl.BlockSpec((D_IN, D_OUT), lambda i: (0, 0)),
            pl.BlockSpec((D_IN, D_OUT), lambda i: (0, 0)),
            pl.BlockSpec((1, D_OUT), lambda i: (0, 0)),
        ],
        out_specs=pl.BlockSpec((BN, D_OUT), lambda i: (i, 0)),
        out_shape=jax.ShapeDtypeStruct((N, D_OUT), jnp.float32),
        compiler_params=pltpu.CompilerParams(
            dimension_semantics=("parallel",),
        ),
    )(self_embs, neigh_embs, wts, wtn, b)
